# baseline (speedup 1.0000x reference)
import functools
import jax
import jax.numpy as jnp
from jax import lax
from jax.experimental import pallas as pl
from jax.experimental.pallas import tpu as pltpu
from jax.experimental.pallas import tpu_sc as plsc

BATCH = 4096
HIST = 200
D = 64
TOTAL = BATCH * HIST

_info = plsc.get_sparse_core_info()
NC, NS, NL = _info.num_cores, _info.num_subcores, _info.num_lanes
NW = NC * NS
B_PER_W = TOTAL // NW
CHUNK = 200
N_CHUNKS = B_PER_W // CHUNK

_mesh = plsc.VectorSubcoreMesh(core_axis_name="c", subcore_axis_name="s")


@functools.partial(
    pl.kernel,
    mesh=_mesh,
    out_type=jax.ShapeDtypeStruct((TOTAL, D), jnp.float32),
    scratch_types=[
        pltpu.VMEM((2, CHUNK, 2 * D), jnp.float32),
        pltpu.VMEM((2, CHUNK, D), jnp.float32),
        pltpu.VMEM((CHUNK,), jnp.int32),
        pltpu.VMEM((CHUNK,), jnp.int32),
        pltpu.VMEM((2, CHUNK), jnp.int32),
        pltpu.SemaphoreType.DMA,
        pltpu.SemaphoreType.DMA,
    ],
)
def _gather_kernel(table_hbm, idx_hbm, out_hbm, pairs_v, rows_v,
                   rowidx0_v, rowidx1_v, par_v, gsem, wsem):
    rowidx_bufs = (rowidx0_v, rowidx1_v)
    wid = lax.axis_index("s") * NC + lax.axis_index("c")
    base = wid * B_PER_W

    def stage_idx(i, b):
        pltpu.sync_copy(
            idx_hbm.at[pl.ds(base + i * CHUNK, CHUNK)], rowidx_bufs[b])

        def grp(g, carry):
            v = rowidx_bufs[b][pl.ds(g * NL, NL)]
            rowidx_bufs[b][pl.ds(g * NL, NL)] = lax.shift_right_logical(v, 1)
            par_v[b, pl.ds(g * NL, NL)] = lax.bitwise_and(v, 1) * D
            return carry

        lax.fori_loop(0, CHUNK // NL, grp, 0, unroll=4)

    def gather_desc(i, b):
        return pltpu.make_async_copy(
            table_hbm.at[rowidx_bufs[b]], pairs_v.at[b], gsem)

    def compact(i, b):
        def grp(g, carry):
            v16 = par_v[b, pl.ds(g * NL, NL)]
            for l in range(NL):
                r = g * NL + l
                h = v16[l]
                for j0 in range(0, D, NL):
                    rows_v[b, r, pl.ds(j0, NL)] = (
                        pairs_v[b, r, pl.ds(h + j0, NL)])
            return carry

        lax.fori_loop(0, CHUNK // NL, grp, 0)

    def write_desc(i, b):
        return pltpu.make_async_copy(
            rows_v.at[b], out_hbm.at[pl.ds(base + i * CHUNK, CHUNK)], wsem)

    stage_idx(0, 0)
    gather_desc(0, 0).start()

    def body(g, carry):
        i0 = 2 * g
        i1 = i0 + 1

        @pl.when(g > 0)
        def _():
            write_desc(i0 - 2, 0).wait()

        stage_idx(i1, 1)
        gather_desc(i1, 1).start()
        gather_desc(i0, 0).wait()
        compact(i0, 0)
        write_desc(i0, 0).start()

        @pl.when(g > 0)
        def _():
            write_desc(i1 - 2, 1).wait()

        @pl.when(i0 + 2 < N_CHUNKS)
        def _():
            stage_idx(i0 + 2, 0)
            gather_desc(i0 + 2, 0).start()

        gather_desc(i1, 1).wait()
        compact(i1, 1)
        write_desc(i1, 1).start()
        return carry

    lax.fori_loop(0, N_CHUNKS // 2, body, 0)
    write_desc(N_CHUNKS - 2, 0).wait()
    write_desc(N_CHUNKS - 1, 1).wait()


def kernel(indices, table):
    idx_flat = indices.reshape(TOTAL).astype(jnp.int32)
    table_pairs = jnp.concatenate([table[0::2], table[1::2]], axis=1)
    out = _gather_kernel(table_pairs, idx_flat)
    return out.reshape(BATCH, HIST, D)


# speedup vs baseline: 9.3800x; 9.3800x over previous
import functools
import jax
import jax.numpy as jnp
from jax import lax
from jax.experimental import pallas as pl
from jax.experimental.pallas import tpu as pltpu
from jax.experimental.pallas import tpu_sc as plsc

BATCH = 4096
HIST = 200
D = 64
TOTAL = BATCH * HIST

_info = plsc.get_sparse_core_info()
NC, NS, NL = _info.num_cores, _info.num_subcores, _info.num_lanes
NW = NC * NS
B_PER_W = TOTAL // NW
CHUNK = 200
N_CHUNKS = B_PER_W // CHUNK

_mesh = plsc.VectorSubcoreMesh(core_axis_name="c", subcore_axis_name="s")


@functools.partial(
    pl.kernel,
    mesh=_mesh,
    out_type=jax.ShapeDtypeStruct((TOTAL, D), jnp.float32),
    scratch_types=[
        pltpu.VMEM((2, CHUNK, 2 * D), jnp.float32),
        pltpu.VMEM((2, CHUNK, D), jnp.float32),
        pltpu.VMEM((CHUNK,), jnp.int32),
        pltpu.VMEM((CHUNK,), jnp.int32),
        pltpu.SemaphoreType.DMA,
        pltpu.SemaphoreType.DMA,
    ],
)
def _gather_kernel(table_hbm, idx_hbm, out_hbm, pairs_v, rows_v,
                   rowidx0_v, rowidx1_v, gsem, wsem):
    rowidx_bufs = (rowidx0_v, rowidx1_v)
    wid = lax.axis_index("s") * NC + lax.axis_index("c")
    base = wid * B_PER_W

    def stage_idx(i, b):
        pltpu.sync_copy(
            idx_hbm.at[pl.ds(base + i * CHUNK, CHUNK)], rowidx_bufs[b])

    def gather_desc(i, b):
        return pltpu.make_async_copy(
            table_hbm.at[rowidx_bufs[b]], pairs_v.at[b], gsem)

    def compact(b):
        def row(r, carry):
            for j0 in range(0, D, NL):
                rows_v[b, r, pl.ds(j0, NL)] = pairs_v[b, r, pl.ds(j0, NL)]
            return carry

        lax.fori_loop(0, CHUNK, row, 0)

    def write_desc(i, b):
        return pltpu.make_async_copy(
            rows_v.at[b], out_hbm.at[pl.ds(base + i * CHUNK, CHUNK)], wsem)

    stage_idx(0, 0)
    gather_desc(0, 0).start()

    def body(g, carry):
        i0 = 2 * g
        i1 = i0 + 1

        @pl.when(g > 0)
        def _():
            write_desc(i0 - 2, 0).wait()

        stage_idx(i1, 1)
        gather_desc(i1, 1).start()
        gather_desc(i0, 0).wait()
        compact(0)
        write_desc(i0, 0).start()

        @pl.when(g > 0)
        def _():
            write_desc(i1 - 2, 1).wait()

        @pl.when(i0 + 2 < N_CHUNKS)
        def _():
            stage_idx(i0 + 2, 0)
            gather_desc(i0 + 2, 0).start()

        gather_desc(i1, 1).wait()
        compact(1)
        write_desc(i1, 1).start()
        return carry

    lax.fori_loop(0, N_CHUNKS // 2, body, 0)
    write_desc(N_CHUNKS - 2, 0).wait()
    write_desc(N_CHUNKS - 1, 1).wait()


def kernel(indices, table):
    idx_flat = indices.reshape(TOTAL).astype(jnp.int32)
    table_pad = jnp.pad(table, ((0, 0), (0, D)))
    out = _gather_kernel(table_pad, idx_flat)
    return out.reshape(BATCH, HIST, D)
